# Initial kernel scaffold; baseline (speedup 1.0000x reference)
import jax
import jax.numpy as jnp
from jax.experimental import pallas as pl


def kernel(x, pos_embedding):
    raise NotImplementedError("write your pallas kernel here")


# speedup vs baseline: 1.0860x; 1.0860x over previous
import jax
import jax.numpy as jnp
from jax.experimental import pallas as pl

_TB = 64


def _add_kernel(x_ref, emb_ref, o_ref, *, s):
    emb = emb_ref[...]
    emb_s = jnp.concatenate([emb] * s, axis=-1)
    o_ref[...] = x_ref[...] + emb_s[None, :, :]


def kernel(x, pos_embedding):
    B, T, S, D = x.shape
    x3 = x.reshape(B, T, S * D)
    from functools import partial

    out = pl.pallas_call(
        partial(_add_kernel, s=S),
        grid=(T // _TB,),
        in_specs=[
            pl.BlockSpec((B, _TB, S * D), lambda t: (0, t, 0)),
            pl.BlockSpec((_TB, D), lambda t: (t, 0)),
        ],
        out_specs=pl.BlockSpec((B, _TB, S * D), lambda t: (0, t, 0)),
        out_shape=jax.ShapeDtypeStruct((B, T, S * D), x.dtype),
    )(x3, pos_embedding)
    return out.reshape(B, T, S, D)


# speedup vs baseline: 1.0867x; 1.0007x over previous
import jax
import jax.numpy as jnp
from jax.experimental import pallas as pl
from jax.experimental.pallas import tpu as pltpu

_TB = 128


def _add_kernel(x_ref, emb_ref, o_ref, *, s):
    emb = emb_ref[...]
    emb_s = jnp.concatenate([emb] * s, axis=-1)
    o_ref[...] = x_ref[...] + emb_s[None, :, :]


def kernel(x, pos_embedding):
    B, T, S, D = x.shape
    x3 = x.reshape(B, T, S * D)
    from functools import partial

    out = pl.pallas_call(
        partial(_add_kernel, s=S),
        grid=(T // _TB,),
        in_specs=[
            pl.BlockSpec((B, _TB, S * D), lambda t: (0, t, 0)),
            pl.BlockSpec((_TB, D), lambda t: (t, 0)),
        ],
        out_specs=pl.BlockSpec((B, _TB, S * D), lambda t: (0, t, 0)),
        out_shape=jax.ShapeDtypeStruct((B, T, S * D), x.dtype),
        compiler_params=pltpu.CompilerParams(
            dimension_semantics=("parallel",),
        ),
    )(x3, pos_embedding)
    return out.reshape(B, T, S, D)
